# baseline (speedup 1.0000x reference)
import functools

import jax
import jax.numpy as jnp
from jax import lax
from jax.experimental import pallas as pl
from jax.experimental.pallas import tpu as pltpu
from jax.experimental.pallas import tpu_sc as plsc

LATENT_DIM = 64
BATCH = 16384
NUM_CLASSES = 1000000

LANE_TILE = 128
TC_TOTAL = (NUM_CLASSES + LANE_TILE - 1) // LANE_TILE
TPW = 245
COLS_PW = TPW * LANE_TILE
CAP = 768
COLSTAGE = 64

_info = plsc.get_sparse_core_info()
_NC, _NS = _info.num_cores, _info.num_subcores
NW = _NC * _NS

_LAST_FULL = TC_TOTAL - (NW - 1) * TPW - 1
_TAIL_COL0 = (TC_TOTAL - 1) * LANE_TILE
_TAIL_W = NUM_CLASSES - _TAIL_COL0

_mesh = plsc.VectorSubcoreMesh(core_axis_name="c", subcore_axis_name="s")


@functools.partial(
    pl.kernel,
    mesh=_mesh,
    out_type=jax.ShapeDtypeStruct((BATCH, LATENT_DIM), jnp.float32),
    scratch_types=[
        pltpu.VMEM((BATCH,), jnp.int32),
        pltpu.VMEM((CAP,), jnp.int32),
        pltpu.VMEM((12, LATENT_DIM, LANE_TILE), jnp.float32),
        pltpu.VMEM((COLSTAGE, LATENT_DIM), jnp.float32),
        pltpu.SMEM((CAP,), jnp.int32),
        pltpu.SMEM((TPW + 1,), jnp.int32),
        pltpu.SMEM((TPW + 1,), jnp.int32),
        pltpu.SemaphoreType.DMA((12,)),
        pltpu.SemaphoreType.DMA,
    ],
    compiler_params=pltpu.CompilerParams(needs_layout_passes=False),
)
def _gather_kernel(
    y_hbm,
    tab_hbm,
    out_hbm,
    yall_v,
    hits_v,
    win_v,
    colstage_v,
    sorted_s,
    off_s,
    cur_s,
    sem_win,
    sem_out,
):
    wid = lax.axis_index("s") * _NC + lax.axis_index("c")
    lo = wid * COLS_PW
    hi = jnp.minimum(lo + COLS_PW, NUM_CLASSES)
    n_full = jnp.where(wid == NW - 1, _LAST_FULL, TPW)
    iota16 = lax.iota(jnp.int32, 16)

    pltpu.sync_copy(y_hbm, yall_v)

    def sel(g, cnt):
        vs, ms, pks, cs = [], [], [], []
        for u in range(4):
            i0 = g * 64 + u * 16
            v = yall_v[pl.ds(i0, 16)]
            m = (v >= lo) & (v < hi)
            vs.append(v)
            ms.append(m)
            pks.append(((v - lo) << 14) | (i0 + iota16))
            cs.append(plsc.cumsum(jnp.where(m, 1, 0).astype(jnp.int32)))
        t = cnt
        for u in range(4):
            plsc.store_scatter(hits_v, [t + cs[u] - 1], pks[u], mask=ms[u])
            t = t + cs[u][15]
        return t

    cnt = lax.fori_loop(0, BATCH // 64, sel, jnp.int32(0))
    ngr = (cnt + 15) // 16

    def zero_t(t, _):
        off_s[t] = 0
        return 0

    lax.fori_loop(0, TPW + 1, zero_t, 0)

    def hist_g(g, _):
        v = hits_v[pl.ds(g * 16, 16)]
        for j in range(16):
            pk = v[j]
            valid = (g * 16 + j) < cnt
            t = jnp.where(valid, pk >> 21, TPW)
            off_s[t] = off_s[t] + jnp.where(valid, 1, 0)
        return 0

    lax.fori_loop(0, ngr, hist_g, 0)

    def pfx(t, acc):
        acc = acc + off_s[t]
        off_s[t] = acc
        return acc

    lax.fori_loop(0, TPW, pfx, jnp.int32(0))

    cur_s[0] = 0

    def curinit(t, _):
        cur_s[t] = off_s[t - 1]
        return 0

    lax.fori_loop(1, TPW + 1, curinit, 0)

    def place_g(g, _):
        v = hits_v[pl.ds(g * 16, 16)]
        for j in range(16):
            pk = v[j]
            valid = (g * 16 + j) < cnt
            t = jnp.where(valid, pk >> 21, TPW)
            slot = cur_s[t]
            cur_s[t] = slot + 1
            sorted_s[slot] = pk
        return 0

    lax.fori_loop(0, ngr, place_g, 0)

    def fire(t_next):
        b = lax.rem(t_next, 12)
        off = pl.multiple_of((wid * TPW + t_next) * LANE_TILE, LANE_TILE)
        pltpu.async_copy(
            tab_hbm.at[:, pl.ds(off, LANE_TILE)], win_v.at[b], sem_win.at[b]
        )

    def process(t, b):
        start = jnp.where(t == 0, 0, off_s[jnp.maximum(t - 1, 0)])
        end = off_s[t]
        winb = win_v.at[b]

        def hit(s, _):
            pk = sorted_s[s]
            ig = pk & (BATCH - 1)
            rel = pk >> 14
            l = jnp.full((16,), rel & (LANE_TILE - 1), jnp.int32)
            kk = s - start
            for c in range(LATENT_DIM // 16):
                vals = plsc.load_gather(winb, [c * 16 + iota16, l])
                colstage_v[kk, pl.ds(c * 16, 16)] = vals
            pltpu.async_copy(colstage_v.at[kk], out_hbm.at[ig], sem_out)
            return 0

        lax.fori_loop(start, end, hit, 0)

        def drain(s, _):
            pltpu.make_async_copy(
                colstage_v.at[0], out_hbm.at[0], sem_out
            ).wait()
            return 0

        lax.fori_loop(start, end, drain, 0)

    for p in range(11):

        @pl.when(p < n_full)
        def _():
            fire(jnp.int32(p))

    def wbody(t, _):
        @pl.when(t + 11 < n_full)
        def _():
            fire(t + 11)

        b = lax.rem(t, 12)
        pltpu.make_async_copy(
            tab_hbm.at[:, pl.ds(0, LANE_TILE)], win_v.at[b], sem_win.at[b]
        ).wait()
        process(t, b)
        return 0

    lax.fori_loop(0, n_full, wbody, 0)

    @pl.when(wid == NW - 1)
    def _():
        b = _LAST_FULL % 12
        for f in range(LATENT_DIM):
            pltpu.async_copy(
                tab_hbm.at[f, pl.ds(_TAIL_COL0, _TAIL_W)],
                win_v.at[b].at[f, pl.ds(0, _TAIL_W)],
                sem_win.at[b],
            )
        for f in range(LATENT_DIM):
            pltpu.make_async_copy(
                tab_hbm.at[f, pl.ds(_TAIL_COL0, _TAIL_W)],
                win_v.at[b].at[f, pl.ds(0, _TAIL_W)],
                sem_win.at[b],
            ).wait()
        process(jnp.int32(_LAST_FULL), b)


def kernel(y, codebook):
    return _gather_kernel(y.astype(jnp.int32), codebook.T)


# speedup vs baseline: 1.1228x; 1.1228x over previous
import functools

import jax
import jax.numpy as jnp
from jax import lax
from jax.experimental import pallas as pl
from jax.experimental.pallas import tpu as pltpu
from jax.experimental.pallas import tpu_sc as plsc

LATENT_DIM = 64
BATCH = 16384
NUM_CLASSES = 1000000

LANE_TILE = 128
TC_TOTAL = (NUM_CLASSES + LANE_TILE - 1) // LANE_TILE
TPW = 245
COLS_PW = TPW * LANE_TILE
CAP = 768
COLSTAGE = 64

_info = plsc.get_sparse_core_info()
_NC, _NS = _info.num_cores, _info.num_subcores
NW = _NC * _NS

_LAST_FULL = TC_TOTAL - (NW - 1) * TPW - 1
_TAIL_COL0 = (TC_TOTAL - 1) * LANE_TILE
_TAIL_W = NUM_CLASSES - _TAIL_COL0

_mesh = plsc.VectorSubcoreMesh(core_axis_name="c", subcore_axis_name="s")


@functools.partial(
    pl.kernel,
    mesh=_mesh,
    out_type=jax.ShapeDtypeStruct((BATCH, LATENT_DIM), jnp.float32),
    scratch_types=[
        pltpu.VMEM((BATCH,), jnp.int32),
        pltpu.VMEM((CAP,), jnp.int32),
        pltpu.VMEM((8, LATENT_DIM, LANE_TILE), jnp.float32),
        pltpu.VMEM((COLSTAGE, LATENT_DIM), jnp.float32),
        pltpu.SMEM((CAP,), jnp.int32),
        pltpu.SMEM((TPW + 1,), jnp.int32),
        pltpu.SMEM((TPW + 1,), jnp.int32),
        pltpu.SemaphoreType.DMA((8,)),
        pltpu.SemaphoreType.DMA,
    ],
    compiler_params=pltpu.CompilerParams(needs_layout_passes=False),
)
def _gather_kernel(
    y_hbm,
    tab_hbm,
    out_hbm,
    yall_v,
    hits_v,
    win_v,
    colstage_v,
    sorted_s,
    off_s,
    cur_s,
    sem_win,
    sem_out,
):
    wid = lax.axis_index("s") * _NC + lax.axis_index("c")
    lo = wid * COLS_PW
    hi = jnp.minimum(lo + COLS_PW, NUM_CLASSES)
    n_full = jnp.where(wid == NW - 1, _LAST_FULL, TPW)
    iota16 = lax.iota(jnp.int32, 16)

    pltpu.sync_copy(y_hbm, yall_v)

    def sel(g, cnt):
        vs, ms, pks, cs = [], [], [], []
        for u in range(4):
            i0 = g * 64 + u * 16
            v = yall_v[pl.ds(i0, 16)]
            m = (v >= lo) & (v < hi)
            vs.append(v)
            ms.append(m)
            pks.append(((v - lo) << 14) | (i0 + iota16))
            cs.append(plsc.cumsum(jnp.where(m, 1, 0).astype(jnp.int32)))
        t = cnt
        for u in range(4):
            plsc.store_scatter(hits_v, [t + cs[u] - 1], pks[u], mask=ms[u])
            t = t + cs[u][15]
        return t

    cnt = lax.fori_loop(0, BATCH // 64, sel, jnp.int32(0))
    ngr = (cnt + 15) // 16

    def zero_t(t, _):
        off_s[t] = 0
        return 0

    lax.fori_loop(0, TPW + 1, zero_t, 0)

    def hist_g(g, _):
        v = hits_v[pl.ds(g * 16, 16)]
        for j in range(16):
            pk = v[j]
            valid = (g * 16 + j) < cnt
            t = jnp.where(valid, pk >> 21, TPW)
            off_s[t] = off_s[t] + jnp.where(valid, 1, 0)
        return 0

    lax.fori_loop(0, ngr, hist_g, 0)

    def pfx(t, acc):
        acc = acc + off_s[t]
        off_s[t] = acc
        return acc

    lax.fori_loop(0, TPW, pfx, jnp.int32(0))

    cur_s[0] = 0

    def curinit(t, _):
        cur_s[t] = off_s[t - 1]
        return 0

    lax.fori_loop(1, TPW + 1, curinit, 0)

    def place_g(g, _):
        v = hits_v[pl.ds(g * 16, 16)]
        for j in range(16):
            pk = v[j]
            valid = (g * 16 + j) < cnt
            t = jnp.where(valid, pk >> 21, TPW)
            slot = cur_s[t]
            cur_s[t] = slot + 1
            sorted_s[slot] = pk
        return 0

    lax.fori_loop(0, ngr, place_g, 0)

    def build(t, nw_):
        prev = jnp.where(t == 0, 0, off_s[jnp.maximum(t - 1, 0)])
        nonempty = off_s[t] > prev
        cur_s[jnp.where(nonempty, nw_, TPW)] = t
        return nw_ + jnp.where(nonempty, 1, 0)

    nwin = lax.fori_loop(0, n_full, build, jnp.int32(0))

    def fire(k):
        b = k & 7
        t_next = cur_s[k]
        off = pl.multiple_of((wid * TPW + t_next) * LANE_TILE, LANE_TILE)
        pltpu.async_copy(
            tab_hbm.at[:, pl.ds(off, LANE_TILE)], win_v.at[b], sem_win.at[b]
        )

    def process(t, b):
        start = jnp.where(t == 0, 0, off_s[jnp.maximum(t - 1, 0)])
        end = off_s[t]
        winb = win_v.at[b]

        def hit(s, _):
            pk = sorted_s[s]
            ig = pk & (BATCH - 1)
            rel = pk >> 14
            l = jnp.full((16,), rel & (LANE_TILE - 1), jnp.int32)
            kk = s - start
            for c in range(LATENT_DIM // 16):
                vals = plsc.load_gather(winb, [c * 16 + iota16, l])
                colstage_v[kk, pl.ds(c * 16, 16)] = vals
            pltpu.async_copy(colstage_v.at[kk], out_hbm.at[ig], sem_out)
            return 0

        lax.fori_loop(start, end, hit, 0)

        def drain(s, _):
            pltpu.make_async_copy(
                colstage_v.at[0], out_hbm.at[0], sem_out
            ).wait()
            return 0

        lax.fori_loop(start, end, drain, 0)

    for p in range(7):

        @pl.when(p < nwin)
        def _():
            fire(jnp.int32(p))

    def wbody(k, _):
        @pl.when(k + 7 < nwin)
        def _():
            fire(k + 7)

        b = k & 7
        pltpu.make_async_copy(
            tab_hbm.at[:, pl.ds(0, LANE_TILE)], win_v.at[b], sem_win.at[b]
        ).wait()
        process(cur_s[k], b)
        return 0

    lax.fori_loop(0, nwin, wbody, 0)

    @pl.when(wid == NW - 1)
    def _():
        b = _LAST_FULL & 7
        for f in range(LATENT_DIM):
            pltpu.async_copy(
                tab_hbm.at[f, pl.ds(_TAIL_COL0, _TAIL_W)],
                win_v.at[b].at[f, pl.ds(0, _TAIL_W)],
                sem_win.at[b],
            )
        for f in range(LATENT_DIM):
            pltpu.make_async_copy(
                tab_hbm.at[f, pl.ds(_TAIL_COL0, _TAIL_W)],
                win_v.at[b].at[f, pl.ds(0, _TAIL_W)],
                sem_win.at[b],
            ).wait()
        process(jnp.int32(_LAST_FULL), b)


def kernel(y, codebook):
    return _gather_kernel(y.astype(jnp.int32), codebook.T)


# speedup vs baseline: 1.1446x; 1.0194x over previous
import functools

import jax
import jax.numpy as jnp
from jax import lax
from jax.experimental import pallas as pl
from jax.experimental.pallas import tpu as pltpu
from jax.experimental.pallas import tpu_sc as plsc

LATENT_DIM = 64
BATCH = 16384
NUM_CLASSES = 1000000

LANE_TILE = 128
TC_TOTAL = (NUM_CLASSES + LANE_TILE - 1) // LANE_TILE
TPW = 245
COLS_PW = TPW * LANE_TILE
CAP = 768
COLSTAGE = 64

_info = plsc.get_sparse_core_info()
_NC, _NS = _info.num_cores, _info.num_subcores
NW = _NC * _NS

_LAST_FULL = TC_TOTAL - (NW - 1) * TPW - 1
_TAIL_COL0 = (TC_TOTAL - 1) * LANE_TILE
_TAIL_W = NUM_CLASSES - _TAIL_COL0

_mesh = plsc.VectorSubcoreMesh(core_axis_name="c", subcore_axis_name="s")


@functools.partial(
    pl.kernel,
    mesh=_mesh,
    out_type=jax.ShapeDtypeStruct((BATCH, LATENT_DIM), jnp.float32),
    scratch_types=[
        pltpu.VMEM((BATCH,), jnp.int32),
        pltpu.VMEM((CAP,), jnp.int32),
        pltpu.VMEM((8, LATENT_DIM, LANE_TILE), jnp.float32),
        pltpu.VMEM((COLSTAGE, LATENT_DIM), jnp.float32),
        pltpu.SMEM((CAP,), jnp.int32),
        pltpu.SMEM((TPW + 1,), jnp.int32),
        pltpu.SMEM((TPW + 1,), jnp.int32),
        pltpu.SMEM((TPW,), jnp.int32),
        pltpu.SemaphoreType.DMA((8,)),
        pltpu.SemaphoreType.DMA,
    ],
    compiler_params=pltpu.CompilerParams(needs_layout_passes=False),
)
def _gather_kernel(
    y_hbm,
    tab_hbm,
    out_hbm,
    yall_v,
    hits_v,
    win_v,
    colstage_v,
    sorted_s,
    off_s,
    cur_s,
    wlist_s,
    sem_win,
    sem_out,
):
    wid = lax.axis_index("s") * _NC + lax.axis_index("c")
    lo = wid * COLS_PW
    hi = jnp.minimum(lo + COLS_PW, NUM_CLASSES)
    n_full = jnp.where(wid == NW - 1, _LAST_FULL, TPW)
    iota16 = lax.iota(jnp.int32, 16)

    pltpu.sync_copy(y_hbm, yall_v)

    def sel(g, cnt):
        vs, ms, pks, cs = [], [], [], []
        for u in range(4):
            i0 = g * 64 + u * 16
            v = yall_v[pl.ds(i0, 16)]
            m = (v >= lo) & (v < hi)
            vs.append(v)
            ms.append(m)
            pks.append(((v - lo) << 14) | (i0 + iota16))
            cs.append(plsc.cumsum(jnp.where(m, 1, 0).astype(jnp.int32)))
        t = cnt
        for u in range(4):
            plsc.store_scatter(hits_v, [t + cs[u] - 1], pks[u], mask=ms[u])
            t = t + cs[u][15]
        return t

    cnt = lax.fori_loop(0, BATCH // 64, sel, jnp.int32(0))
    ngr = (cnt + 15) // 16

    def zero_t(t, _):
        off_s[t] = 0
        return 0

    lax.fori_loop(0, TPW + 1, zero_t, 0)

    def hist_g(g, _):
        v = hits_v[pl.ds(g * 16, 16)]
        for j in range(16):
            pk = v[j]
            valid = (g * 16 + j) < cnt
            t = jnp.where(valid, pk >> 21, TPW)
            off_s[t] = off_s[t] + jnp.where(valid, 1, 0)
        return 0

    lax.fori_loop(0, ngr, hist_g, 0)

    def pfx(t, acc):
        acc = acc + off_s[t]
        off_s[t] = acc
        return acc

    lax.fori_loop(0, TPW, pfx, jnp.int32(0))

    def build(t, nw_):
        prev = jnp.where(t == 0, 0, off_s[jnp.maximum(t - 1, 0)])
        nonempty = off_s[t] > prev
        wlist_s[jnp.where(nonempty, nw_, TPW - 1)] = t
        return nw_ + jnp.where(nonempty, 1, 0)

    nwin = lax.fori_loop(0, n_full, build, jnp.int32(0))

    def fire(k):
        b = k & 7
        t_next = wlist_s[k]
        off = pl.multiple_of((wid * TPW + t_next) * LANE_TILE, LANE_TILE)
        pltpu.async_copy(
            tab_hbm.at[:, pl.ds(off, LANE_TILE)], win_v.at[b], sem_win.at[b]
        )

    def process(t, b):
        start = jnp.where(t == 0, 0, off_s[jnp.maximum(t - 1, 0)])
        end = off_s[t]
        winb = win_v.at[b]

        def hit(s, _):
            pk = sorted_s[s]
            ig = pk & (BATCH - 1)
            rel = pk >> 14
            l = jnp.full((16,), rel & (LANE_TILE - 1), jnp.int32)
            kk = s - start
            for c in range(LATENT_DIM // 16):
                vals = plsc.load_gather(winb, [c * 16 + iota16, l])
                colstage_v[kk, pl.ds(c * 16, 16)] = vals
            pltpu.async_copy(colstage_v.at[kk], out_hbm.at[ig], sem_out)
            return 0

        lax.fori_loop(start, end, hit, 0)

        def drain(s, _):
            pltpu.make_async_copy(
                colstage_v.at[0], out_hbm.at[0], sem_out
            ).wait()
            return 0

        lax.fori_loop(start, end, drain, 0)

    for p in range(7):

        @pl.when(p < nwin)
        def _():
            fire(jnp.int32(p))

    cur_s[0] = 0

    def curinit(t, _):
        cur_s[t] = off_s[t - 1]
        return 0

    lax.fori_loop(1, TPW + 1, curinit, 0)

    def place_g(g, _):
        v = hits_v[pl.ds(g * 16, 16)]
        for j in range(16):
            pk = v[j]
            valid = (g * 16 + j) < cnt
            t = jnp.where(valid, pk >> 21, TPW)
            slot = cur_s[t]
            cur_s[t] = slot + 1
            sorted_s[slot] = pk
        return 0

    lax.fori_loop(0, ngr, place_g, 0)


    def wbody(k, _):
        @pl.when(k + 7 < nwin)
        def _():
            fire(k + 7)

        b = k & 7
        pltpu.make_async_copy(
            tab_hbm.at[:, pl.ds(0, LANE_TILE)], win_v.at[b], sem_win.at[b]
        ).wait()
        process(wlist_s[k], b)
        return 0

    lax.fori_loop(0, nwin, wbody, 0)

    @pl.when(wid == NW - 1)
    def _():
        b = _LAST_FULL & 7
        for f in range(LATENT_DIM):
            pltpu.async_copy(
                tab_hbm.at[f, pl.ds(_TAIL_COL0, _TAIL_W)],
                win_v.at[b].at[f, pl.ds(0, _TAIL_W)],
                sem_win.at[b],
            )
        for f in range(LATENT_DIM):
            pltpu.make_async_copy(
                tab_hbm.at[f, pl.ds(_TAIL_COL0, _TAIL_W)],
                win_v.at[b].at[f, pl.ds(0, _TAIL_W)],
                sem_win.at[b],
            ).wait()
        process(jnp.int32(_LAST_FULL), b)


def kernel(y, codebook):
    return _gather_kernel(y.astype(jnp.int32), codebook.T)
